# Initial kernel scaffold; baseline (speedup 1.0000x reference)
import jax
import jax.numpy as jnp
from jax.experimental import pallas as pl


def kernel(x, embeddings):
    raise NotImplementedError("write your pallas kernel here")


# speedup vs baseline: 15.8585x; 15.8585x over previous
import functools

import jax
import jax.numpy as jnp
from jax import lax
from jax.experimental import pallas as pl
from jax.experimental.pallas import tpu as pltpu
from jax.experimental.pallas import tpu_sc as plsc

VOCAB_P1 = 100001
EMB = 32
LEN_SEQ = 4
MAP_H = 16
MAP_W = 16
SEQ_SIZE = 20
BATCH = 32

PAIRS = BATCH * LEN_SEQ
SEGS_PER_PAIR = MAP_H * MAP_W
IDX_PER_PAIR = SEGS_PER_PAIR * SEQ_SIZE
NW = 32
PAIRS_PER_W = PAIRS // NW
IDX_ROW = 128
ROWS_PER_CHUNK = 640
SEGS_PER_CHUNK = ROWS_PER_CHUNK // SEQ_SIZE
CHUNKS = IDX_PER_PAIR // ROWS_PER_CHUNK
GATHERS_PER_CHUNK = ROWS_PER_CHUNK // IDX_ROW


def _sc_body(x_hbm, emb_hbm, out_hbm, idx_v, rows_v, acc_v, sem):
    wid = lax.axis_index("s") * 2 + lax.axis_index("c")
    iota = lax.iota(jnp.int32, 16)
    sc0 = iota * SEGS_PER_PAIR
    sc1 = sc0 + 16 * SEGS_PER_PAIR

    def seg_body(s, c):
        base = s * SEQ_SIZE
        a0 = rows_v[base, pl.ds(0, 16)]
        a1 = rows_v[base, pl.ds(16, 16)]
        for k in range(1, SEQ_SIZE):
            a0 = a0 + rows_v[base + k, pl.ds(0, 16)]
            a1 = a1 + rows_v[base + k, pl.ds(16, 16)]
        seg = c * SEGS_PER_CHUNK + s
        plsc.store_scatter(acc_v, [sc0 + seg], a0)
        plsc.store_scatter(acc_v, [sc1 + seg], a1)
        return c

    def chunk_body(c, _):
        copies = []
        for j in range(GATHERS_PER_CHUNK):
            cp = pltpu.make_async_copy(
                emb_hbm.at[idx_v.at[c * GATHERS_PER_CHUNK + j]],
                rows_v.at[pl.ds(j * IDX_ROW, IDX_ROW)],
                sem,
            )
            cp.start()
            copies.append(cp)
        for cp in copies:
            cp.wait()
        lax.fori_loop(0, SEGS_PER_CHUNK, seg_body, c)
        return 0

    def pair_body(pi, _):
        p = wid * PAIRS_PER_W + pi
        pltpu.sync_copy(x_hbm.at[p], idx_v)
        lax.fori_loop(0, CHUNKS, chunk_body, 0)
        pltpu.sync_copy(acc_v, out_hbm.at[p])
        return 0

    lax.fori_loop(0, PAIRS_PER_W, pair_body, 0)


@functools.partial(jax.jit, static_argnames=())
def kernel(x, embeddings):
    x3 = x.astype(jnp.int32).reshape(PAIRS, IDX_PER_PAIR // IDX_ROW, IDX_ROW)
    mesh = plsc.VectorSubcoreMesh(core_axis_name="c", subcore_axis_name="s")
    out = pl.kernel(
        _sc_body,
        mesh=mesh,
        compiler_params=pltpu.CompilerParams(
            needs_layout_passes=False, use_tc_tiling_on_sc=False
        ),
        out_type=jax.ShapeDtypeStruct((PAIRS, EMB * SEGS_PER_PAIR), jnp.float32),
        scratch_types=[
            pltpu.VMEM((IDX_PER_PAIR // IDX_ROW, IDX_ROW), jnp.int32),
            pltpu.VMEM((ROWS_PER_CHUNK, EMB), jnp.float32),
            pltpu.VMEM((EMB * SEGS_PER_PAIR,), jnp.float32),
            pltpu.SemaphoreType.DMA,
        ],
    )(x3, embeddings)
    return out.reshape(BATCH, LEN_SEQ * EMB, MAP_H, MAP_W)


# speedup vs baseline: 18.1755x; 1.1461x over previous
import functools

import jax
import jax.numpy as jnp
from jax import lax
from jax.experimental import pallas as pl
from jax.experimental.pallas import tpu as pltpu
from jax.experimental.pallas import tpu_sc as plsc

VOCAB_P1 = 100001
EMB = 32
LEN_SEQ = 4
MAP_H = 16
MAP_W = 16
SEQ_SIZE = 20
BATCH = 32

PAIRS = BATCH * LEN_SEQ
SEGS_PER_PAIR = MAP_H * MAP_W
IDX_PER_PAIR = SEGS_PER_PAIR * SEQ_SIZE
NW = 32
PAIRS_PER_W = PAIRS // NW
IDX_ROW = 128
ROWS_PER_CHUNK = 640
SEGS_PER_CHUNK = ROWS_PER_CHUNK // SEQ_SIZE
CHUNKS = IDX_PER_PAIR // ROWS_PER_CHUNK
GATHERS_PER_CHUNK = ROWS_PER_CHUNK // IDX_ROW


def _tree_sum(vals):
    while len(vals) > 1:
        nxt = [vals[i] + vals[i + 1] for i in range(0, len(vals) - 1, 2)]
        if len(vals) % 2:
            nxt.append(vals[-1])
        vals = nxt
    return vals[0]


def _sc_body(x_hbm, emb_hbm, out_hbm, idx_v, rows_v, acc_v, sem):
    wid = lax.axis_index("s") * 2 + lax.axis_index("c")
    iota = lax.iota(jnp.int32, 16)
    sc0 = iota * SEGS_PER_PAIR
    sc1 = sc0 + 16 * SEGS_PER_PAIR

    def start_chunk(c, buf):
        cps = []
        for j in range(GATHERS_PER_CHUNK):
            cp = pltpu.make_async_copy(
                emb_hbm.at[idx_v.at[c * GATHERS_PER_CHUNK + j]],
                rows_v.at[buf, pl.ds(j * IDX_ROW, IDX_ROW)],
                sem,
            )
            cp.start()
            cps.append(cp)
        return cps

    def pair_body(pi, _):
        p = wid * PAIRS_PER_W + pi
        pltpu.sync_copy(x_hbm.at[p], idx_v)
        pend = start_chunk(0, 0)
        for c in range(CHUNKS):
            cur = c % 2
            for cp in pend:
                cp.wait()
            if c + 1 < CHUNKS:
                pend = start_chunk(c + 1, 1 - cur)

            @plsc.parallel_loop(0, SEGS_PER_CHUNK, unroll=2)
            def seg_body(s, cur=cur, c=c):
                base = s * SEQ_SIZE
                a0 = _tree_sum(
                    [rows_v[cur, base + k, pl.ds(0, 16)] for k in range(SEQ_SIZE)]
                )
                a1 = _tree_sum(
                    [rows_v[cur, base + k, pl.ds(16, 16)] for k in range(SEQ_SIZE)]
                )
                seg = c * SEGS_PER_CHUNK + s
                plsc.store_scatter(acc_v, [sc0 + seg], a0)
                plsc.store_scatter(acc_v, [sc1 + seg], a1)

        pltpu.sync_copy(acc_v, out_hbm.at[p])
        return 0

    lax.fori_loop(0, PAIRS_PER_W, pair_body, 0)


@functools.partial(jax.jit, static_argnames=())
def kernel(x, embeddings):
    x3 = x.astype(jnp.int32).reshape(PAIRS, IDX_PER_PAIR // IDX_ROW, IDX_ROW)
    mesh = plsc.VectorSubcoreMesh(core_axis_name="c", subcore_axis_name="s")
    out = pl.kernel(
        _sc_body,
        mesh=mesh,
        compiler_params=pltpu.CompilerParams(
            needs_layout_passes=False, use_tc_tiling_on_sc=False
        ),
        out_type=jax.ShapeDtypeStruct((PAIRS, EMB * SEGS_PER_PAIR), jnp.float32),
        scratch_types=[
            pltpu.VMEM((IDX_PER_PAIR // IDX_ROW, IDX_ROW), jnp.int32),
            pltpu.VMEM((2, ROWS_PER_CHUNK, EMB), jnp.float32),
            pltpu.VMEM((EMB * SEGS_PER_PAIR,), jnp.float32),
            pltpu.SemaphoreType.DMA,
        ],
    )(x3, embeddings)
    return out.reshape(BATCH, LEN_SEQ * EMB, MAP_H, MAP_W)
